# baseline (speedup 1.0000x reference)
import jax
import jax.numpy as jnp
from jax.experimental import pallas as pl
from jax.experimental.pallas import tpu as pltpu

B = 8192
OBS = 1024
ACT = 32
M = 16
HID = 256
EPS_W = 1e-06

BM = 2048
TM = 512

_DN = (((1,), (1,)), ((), ()))


def _dot_t(x, w, precision=None):
    return jax.lax.dot_general(x, w, _DN, precision=precision,
                               preferred_element_type=jnp.float32)


def _fused_kernel(obs_ref, w1_ref, b1_ref, w2_ref, b2_ref, wmu_ref, bmu_ref,
                  lsd_ref, v1_ref, vb1_ref, v2_ref, vb2_ref, v3_ref, vb3_ref,
                  cen_ref, sm_ref, sv_ref,
                  mu_ref, ls_ref, val_ref, idx_ref,
                  cw_ref, cb_ref):
    @pl.when(pl.program_id(0) == 0)
    def _fold():
        scale = 1.0 / jnp.sqrt(sv_ref[...] + EPS_W)
        cw = cen_ref[...] * scale
        mm = sm_ref[...] * scale
        cw_ref[...] = cw
        cb_ref[...] = (jnp.sum(cw * cw, axis=1)
                       - 2.0 * jnp.sum(cw * mm, axis=1)
                       + jnp.sum(mm * mm))

    for t in range(BM // TM):
        rows = pl.ds(t * TM, TM)
        obs = obs_ref[rows, :]
        h1 = jnp.tanh(_dot_t(obs, w1_ref[...]) + b1_ref[...])
        feat = jnp.tanh(_dot_t(h1, w2_ref[...]) + b2_ref[...])

        scores = cb_ref[...] - 2.0 * _dot_t(
            feat, cw_ref[...], precision=jax.lax.Precision.HIGHEST)
        lane = jax.lax.broadcasted_iota(jnp.int32, (TM, M), 1)
        best_i = jnp.argmin(scores, axis=1)[:, None].astype(jnp.int32)
        onehot = (lane == best_i).astype(jnp.float32)

        mu_all = _dot_t(feat, wmu_ref[...])
        grp = jax.lax.broadcasted_iota(jnp.int32, (TM, M * ACT), 1) // ACT
        acc = jnp.where(grp == best_i, mu_all, 0.0)
        w = M * ACT
        while w > ACT:
            w //= 2
            acc = acc[:, :w] + acc[:, w:2 * w]
        mu = acc + jnp.dot(onehot, bmu_ref[...],
                           preferred_element_type=jnp.float32)
        ls = jnp.dot(onehot, lsd_ref[...],
                     preferred_element_type=jnp.float32)

        v = jnp.tanh(_dot_t(feat, v1_ref[...]) + vb1_ref[...])
        v = jnp.tanh(_dot_t(v, v2_ref[...]) + vb2_ref[...])
        val = jnp.sum(v * v3_ref[...], axis=1, keepdims=True) + vb3_ref[...]

        mu_ref[rows, :] = mu
        ls_ref[rows, :] = ls
        val_ref[rows, :] = val
        idx_ref[rows, :] = best_i


@jax.jit
def kernel(obs, W1, b1, W2, b2, Wout, bout, Wmu, bmu, log_std,
           V1, Vb1, V2, Vb2, V3, Vb3, centers, stats_mean, stats_var):
    del Wout, bout
    Wmu2 = Wmu.reshape(M * ACT, HID)
    v3_row = V3.reshape(1, HID)
    vb3_row = Vb3.reshape(1, 1)

    grid = (B // BM,)
    row_spec = pl.BlockSpec((BM, OBS), lambda i: (i, 0))
    full = lambda shape: pl.BlockSpec(shape, lambda i: (0,) * len(shape))

    mu, ls, val, idx = pl.pallas_call(
        _fused_kernel,
        grid=grid,
        in_specs=[
            row_spec,
            full((HID, OBS)), full((HID,)),
            full((HID, HID)), full((HID,)),
            full((M * ACT, HID)), full((M, ACT)),
            full((M, ACT)),
            full((HID, HID)), full((HID,)),
            full((HID, HID)), full((HID,)),
            full((1, HID)), full((1, 1)),
            full((M, HID)), full((HID,)), full((HID,)),
        ],
        out_specs=[
            pl.BlockSpec((BM, ACT), lambda i: (i, 0)),
            pl.BlockSpec((BM, ACT), lambda i: (i, 0)),
            pl.BlockSpec((BM, 1), lambda i: (i, 0)),
            pl.BlockSpec((BM, 1), lambda i: (i, 0)),
        ],
        out_shape=[
            jax.ShapeDtypeStruct((B, ACT), jnp.float32),
            jax.ShapeDtypeStruct((B, ACT), jnp.float32),
            jax.ShapeDtypeStruct((B, 1), jnp.float32),
            jax.ShapeDtypeStruct((B, 1), jnp.int32),
        ],
        scratch_shapes=[
            pltpu.VMEM((M, HID), jnp.float32),
            pltpu.VMEM((M,), jnp.float32),
        ],
    )(obs, W1, b1, W2, b2, Wmu2, bmu, log_std,
      V1, Vb1, V2, Vb2, v3_row, vb3_row, centers, stats_mean, stats_var)

    return (mu, ls, val[:, 0], idx[:, 0])


# speedup vs baseline: 1.0610x; 1.0610x over previous
import jax
import jax.numpy as jnp
from jax.experimental import pallas as pl
from jax.experimental.pallas import tpu as pltpu

B = 8192
OBS = 1024
ACT = 32
M = 16
HID = 256
EPS_W = 1e-06

BM = 2048
TM = 1024

_DN = (((1,), (1,)), ((), ()))


def _dot_t(x, w, precision=None):
    return jax.lax.dot_general(x, w, _DN, precision=precision,
                               preferred_element_type=jnp.float32)


def _fused_kernel(obs_ref, w1_ref, b1_ref, w2_ref, b2_ref, wmu_ref, bmu_ref,
                  lsd_ref, v1_ref, vb1_ref, v2_ref, vb2_ref, v3_ref, vb3_ref,
                  cen_ref, sm_ref, sv_ref,
                  mu_ref, ls_ref, val_ref, idx_ref,
                  cw_ref, cb_ref):
    @pl.when(pl.program_id(0) == 0)
    def _fold():
        scale = 1.0 / jnp.sqrt(sv_ref[...] + EPS_W)
        cw = cen_ref[...] * scale
        mm = sm_ref[...] * scale
        cw_ref[...] = cw
        cb_ref[...] = (jnp.sum(cw * cw, axis=1)
                       - 2.0 * jnp.sum(cw * mm, axis=1)
                       + jnp.sum(mm * mm))

    for t in range(BM // TM):
        rows = pl.ds(t * TM, TM)
        obs = obs_ref[rows, :]
        h1 = jnp.tanh(_dot_t(obs, w1_ref[...]) + b1_ref[...])
        feat = jnp.tanh(_dot_t(h1, w2_ref[...]) + b2_ref[...])

        scores = cb_ref[...] - 2.0 * _dot_t(
            feat, cw_ref[...], precision=jax.lax.Precision.HIGHEST)
        lane = jax.lax.broadcasted_iota(jnp.int32, (TM, M), 1)
        best_i = jnp.argmin(scores, axis=1)[:, None].astype(jnp.int32)
        onehot = (lane == best_i).astype(jnp.float32)

        mu_all = _dot_t(feat, wmu_ref[...])
        grp = jax.lax.broadcasted_iota(jnp.int32, (TM, M * ACT), 1) // ACT
        acc = jnp.where(grp == best_i, mu_all, 0.0)
        w = M * ACT
        while w > ACT:
            w //= 2
            acc = acc[:, :w] + acc[:, w:2 * w]
        mu = acc + jnp.dot(onehot, bmu_ref[...],
                           preferred_element_type=jnp.float32)
        ls = jnp.dot(onehot, lsd_ref[...],
                     preferred_element_type=jnp.float32)

        v = jnp.tanh(_dot_t(feat, v1_ref[...]) + vb1_ref[...])
        v = jnp.tanh(_dot_t(v, v2_ref[...]) + vb2_ref[...])
        val = jnp.sum(v * v3_ref[...], axis=1, keepdims=True) + vb3_ref[...]

        mu_ref[rows, :] = mu
        ls_ref[rows, :] = ls
        val_ref[rows, :] = val
        idx_ref[rows, :] = best_i


@jax.jit
def kernel(obs, W1, b1, W2, b2, Wout, bout, Wmu, bmu, log_std,
           V1, Vb1, V2, Vb2, V3, Vb3, centers, stats_mean, stats_var):
    del Wout, bout
    Wmu2 = Wmu.reshape(M * ACT, HID)
    v3_row = V3.reshape(1, HID)
    vb3_row = Vb3.reshape(1, 1)

    grid = (B // BM,)
    row_spec = pl.BlockSpec((BM, OBS), lambda i: (i, 0))
    full = lambda shape: pl.BlockSpec(shape, lambda i: (0,) * len(shape))

    mu, ls, val, idx = pl.pallas_call(
        _fused_kernel,
        grid=grid,
        in_specs=[
            row_spec,
            full((HID, OBS)), full((HID,)),
            full((HID, HID)), full((HID,)),
            full((M * ACT, HID)), full((M, ACT)),
            full((M, ACT)),
            full((HID, HID)), full((HID,)),
            full((HID, HID)), full((HID,)),
            full((1, HID)), full((1, 1)),
            full((M, HID)), full((HID,)), full((HID,)),
        ],
        out_specs=[
            pl.BlockSpec((BM, ACT), lambda i: (i, 0)),
            pl.BlockSpec((BM, ACT), lambda i: (i, 0)),
            pl.BlockSpec((BM, 1), lambda i: (i, 0)),
            pl.BlockSpec((BM, 1), lambda i: (i, 0)),
        ],
        out_shape=[
            jax.ShapeDtypeStruct((B, ACT), jnp.float32),
            jax.ShapeDtypeStruct((B, ACT), jnp.float32),
            jax.ShapeDtypeStruct((B, 1), jnp.float32),
            jax.ShapeDtypeStruct((B, 1), jnp.int32),
        ],
        scratch_shapes=[
            pltpu.VMEM((M, HID), jnp.float32),
            pltpu.VMEM((M,), jnp.float32),
        ],
    )(obs, W1, b1, W2, b2, Wmu2, bmu, log_std,
      V1, Vb1, V2, Vb2, v3_row, vb3_row, centers, stats_mean, stats_var)

    return (mu, ls, val[:, 0], idx[:, 0])


# speedup vs baseline: 1.1164x; 1.0522x over previous
import jax
import jax.numpy as jnp
from jax.experimental import pallas as pl
from jax.experimental.pallas import tpu as pltpu

B = 8192
OBS = 1024
ACT = 32
M = 16
HID = 256
EPS_W = 1e-06

BM = 2048
TM = 2048

_DN = (((1,), (1,)), ((), ()))


def _dot_t(x, w, precision=None):
    return jax.lax.dot_general(x, w, _DN, precision=precision,
                               preferred_element_type=jnp.float32)


def _fused_kernel(obs_ref, w1_ref, b1_ref, w2_ref, b2_ref, wmu_ref, bmu_ref,
                  lsd_ref, v1_ref, vb1_ref, v2_ref, vb2_ref, v3_ref, vb3_ref,
                  cen_ref, sm_ref, sv_ref,
                  mu_ref, ls_ref, val_ref, idx_ref,
                  cw_ref, cb_ref):
    @pl.when(pl.program_id(0) == 0)
    def _fold():
        scale = 1.0 / jnp.sqrt(sv_ref[...] + EPS_W)
        cw = cen_ref[...] * scale
        mm = sm_ref[...] * scale
        cw_ref[...] = cw
        cb_ref[...] = (jnp.sum(cw * cw, axis=1)
                       - 2.0 * jnp.sum(cw * mm, axis=1)
                       + jnp.sum(mm * mm))

    for t in range(BM // TM):
        rows = pl.ds(t * TM, TM)
        obs = obs_ref[rows, :]
        h1 = jnp.tanh(_dot_t(obs, w1_ref[...]) + b1_ref[...])
        feat = jnp.tanh(_dot_t(h1, w2_ref[...]) + b2_ref[...])

        scores = cb_ref[...] - 2.0 * _dot_t(
            feat, cw_ref[...], precision=jax.lax.Precision.HIGHEST)
        lane = jax.lax.broadcasted_iota(jnp.int32, (TM, M), 1)
        best_i = jnp.argmin(scores, axis=1)[:, None].astype(jnp.int32)
        onehot = (lane == best_i).astype(jnp.float32)

        mu_all = _dot_t(feat, wmu_ref[...])
        grp = jax.lax.broadcasted_iota(jnp.int32, (TM, M * ACT), 1) // ACT
        acc = jnp.where(grp == best_i, mu_all, 0.0)
        w = M * ACT
        while w > ACT:
            w //= 2
            acc = acc[:, :w] + acc[:, w:2 * w]
        mu = acc + jnp.dot(onehot, bmu_ref[...],
                           preferred_element_type=jnp.float32)
        ls = jnp.dot(onehot, lsd_ref[...],
                     preferred_element_type=jnp.float32)

        v = jnp.tanh(_dot_t(feat, v1_ref[...]) + vb1_ref[...])
        v = jnp.tanh(_dot_t(v, v2_ref[...]) + vb2_ref[...])
        val = jnp.sum(v * v3_ref[...], axis=1, keepdims=True) + vb3_ref[...]

        mu_ref[rows, :] = mu
        ls_ref[rows, :] = ls
        val_ref[rows, :] = val
        idx_ref[rows, :] = best_i


@jax.jit
def kernel(obs, W1, b1, W2, b2, Wout, bout, Wmu, bmu, log_std,
           V1, Vb1, V2, Vb2, V3, Vb3, centers, stats_mean, stats_var):
    del Wout, bout
    Wmu2 = Wmu.reshape(M * ACT, HID)
    v3_row = V3.reshape(1, HID)
    vb3_row = Vb3.reshape(1, 1)

    grid = (B // BM,)
    row_spec = pl.BlockSpec((BM, OBS), lambda i: (i, 0))
    full = lambda shape: pl.BlockSpec(shape, lambda i: (0,) * len(shape))

    mu, ls, val, idx = pl.pallas_call(
        _fused_kernel,
        grid=grid,
        in_specs=[
            row_spec,
            full((HID, OBS)), full((HID,)),
            full((HID, HID)), full((HID,)),
            full((M * ACT, HID)), full((M, ACT)),
            full((M, ACT)),
            full((HID, HID)), full((HID,)),
            full((HID, HID)), full((HID,)),
            full((1, HID)), full((1, 1)),
            full((M, HID)), full((HID,)), full((HID,)),
        ],
        out_specs=[
            pl.BlockSpec((BM, ACT), lambda i: (i, 0)),
            pl.BlockSpec((BM, ACT), lambda i: (i, 0)),
            pl.BlockSpec((BM, 1), lambda i: (i, 0)),
            pl.BlockSpec((BM, 1), lambda i: (i, 0)),
        ],
        out_shape=[
            jax.ShapeDtypeStruct((B, ACT), jnp.float32),
            jax.ShapeDtypeStruct((B, ACT), jnp.float32),
            jax.ShapeDtypeStruct((B, 1), jnp.float32),
            jax.ShapeDtypeStruct((B, 1), jnp.int32),
        ],
        scratch_shapes=[
            pltpu.VMEM((M, HID), jnp.float32),
            pltpu.VMEM((M,), jnp.float32),
        ],
    )(obs, W1, b1, W2, b2, Wmu2, bmu, log_std,
      V1, Vb1, V2, Vb2, v3_row, vb3_row, centers, stats_mean, stats_var)

    return (mu, ls, val[:, 0], idx[:, 0])
